# Initial kernel scaffold; baseline (speedup 1.0000x reference)
import jax
import jax.numpy as jnp
from jax.experimental import pallas as pl


def kernel(x, edge_index, W_lift, b_lift, W_conv, b_conv, W_out, b_out):
    raise NotImplementedError("write your pallas kernel here")


# speedup vs baseline: 84.7041x; 84.7041x over previous
import functools

import jax
import jax.numpy as jnp
from jax import lax
from jax.experimental import pallas as pl
from jax.experimental.pallas import tpu as pltpu
from jax.experimental.pallas import tpu_sc as plsc

N = 100000
E = 6400000
R = 100
NR = E // R
NC, NS = 2, 16
NW = NC * NS
RPW = NR // NW
K = 16
T = RPW // K
D = 16
NPAD = 100352
SL = NPAD // NS
FCH = SL // 4

_mesh = plsc.VectorSubcoreMesh(
    core_axis_name="c", subcore_axis_name="s", num_cores=NC, num_subcores=NS)


@functools.partial(
    pl.kernel,
    out_type=jax.ShapeDtypeStruct((NC, NPAD), jnp.float32),
    mesh=_mesh,
    scratch_types=[
        pltpu.VMEM((K, R), jnp.int32),
        pltpu.VMEM((112,), jnp.float32),
        pltpu.VMEM((SL,), jnp.float32),
        pltpu.VMEM_SHARED((NPAD,), jnp.float32),
        pltpu.SemaphoreType.DMA,
    ],
)
def _deg_kernel(dst_hbm, deg_out, ibuf, ones, tbuf, deg_sp, sem):
    c = lax.axis_index("c")
    s = lax.axis_index("s")
    w = s * NC + c
    z = jnp.zeros((16,), jnp.float32)
    o = jnp.ones((16,), jnp.float32)
    for i in range(7):
        ones[pl.ds(i * 16, 16)] = o

    def zb(i, _):
        tbuf[pl.ds(i * 16, 16)] = z
        return _
    lax.fori_loop(0, SL // 16, zb, 0)
    pltpu.sync_copy(tbuf, deg_sp.at[pl.ds(s * SL, SL)])
    plsc.subcore_barrier()

    def step(t, _):
        pltpu.sync_copy(dst_hbm.at[pl.ds(w * RPW + t * K, K)], ibuf)

        def fire(j, _):
            pltpu.async_copy(ones.at[pl.ds(0, R)],
                             deg_sp.at[ibuf.at[j]], sem, add=True)
            return _
        lax.fori_loop(0, K, fire, 0)

        def drain(j, _):
            pltpu.make_async_copy(ones.at[pl.ds(0, R)],
                                  deg_sp.at[ibuf.at[j]], sem).wait()
            return _
        lax.fori_loop(0, K, drain, 0)
        return _
    lax.fori_loop(0, T, step, 0)

    plsc.subcore_barrier()
    pltpu.sync_copy(deg_sp.at[pl.ds(s * SL, SL)], tbuf)
    pltpu.sync_copy(tbuf, deg_out.at[c, pl.ds(s * SL, SL)])


@functools.partial(
    pl.kernel,
    out_type=jax.ShapeDtypeStruct((NC, NPAD, D), jnp.float32),
    mesh=_mesh,
    compiler_params=pltpu.CompilerParams(use_tc_tiling_on_sc=False),
    scratch_types=[
        pltpu.VMEM((K, R), jnp.int32),
        pltpu.VMEM((K, R), jnp.int32),
        pltpu.VMEM((K * R, D), jnp.float32),
        pltpu.VMEM_SHARED((NPAD, D), jnp.float32),
        pltpu.SemaphoreType.DMA,
        pltpu.SemaphoreType.DMA,
    ],
)
def _msg_kernel(src_hbm, dst_hbm, g_hbm, acc_out,
                isrc, idst, rows, acc_sp, sem_g, sem_s):
    c = lax.axis_index("c")
    s = lax.axis_index("s")
    w = s * NC + c
    z = jnp.zeros((16,), jnp.float32)

    def zb(i, _):
        rows[i, :] = z
        return _
    lax.fori_loop(0, FCH, zb, 0)
    for q in range(4):
        pltpu.sync_copy(rows.at[pl.ds(0, FCH)],
                        acc_sp.at[pl.ds(s * SL + q * FCH, FCH)])
    plsc.subcore_barrier()

    def step(t, _):
        base = w * RPW + t * K
        pltpu.sync_copy(src_hbm.at[pl.ds(base, K)], isrc)
        pltpu.sync_copy(dst_hbm.at[pl.ds(base, K)], idst)

        def fire_g(j, _):
            pltpu.async_copy(g_hbm.at[isrc.at[j]],
                             rows.at[pl.ds(j * R, R)], sem_g)
            return _
        lax.fori_loop(0, K, fire_g, 0)

        def fire_s(j, _):
            pltpu.make_async_copy(g_hbm.at[isrc.at[j]],
                                  rows.at[pl.ds(j * R, R)], sem_g).wait()
            pltpu.async_copy(rows.at[pl.ds(j * R, R)],
                             acc_sp.at[idst.at[j]], sem_s, add=True)
            return _
        lax.fori_loop(0, K, fire_s, 0)

        def drain_s(j, _):
            pltpu.make_async_copy(rows.at[pl.ds(j * R, R)],
                                  acc_sp.at[idst.at[j]], sem_s).wait()
            return _
        lax.fori_loop(0, K, drain_s, 0)
        return _
    lax.fori_loop(0, T, step, 0)

    plsc.subcore_barrier()
    for q in range(4):
        pltpu.sync_copy(acc_sp.at[pl.ds(s * SL + q * FCH, FCH)],
                        rows.at[pl.ds(0, FCH)])
        pltpu.sync_copy(rows.at[pl.ds(0, FCH)],
                        acc_out.at[c, pl.ds(s * SL + q * FCH, FCH)])


BN = 8192
_GRID = (N + BN - 1) // BN


def _lift(x, wl_pad, bl_pad, wc_pad):
    def body(x_ref, wl_ref, bl_ref, wc_ref, h2_ref):
        h = jnp.tanh(jnp.dot(x_ref[...], wl_ref[...],
                             preferred_element_type=jnp.float32) + bl_ref[...])
        h2_ref[...] = jnp.dot(h, wc_ref[...],
                              preferred_element_type=jnp.float32)
    return pl.pallas_call(
        body,
        grid=(_GRID,),
        in_specs=[
            pl.BlockSpec((BN, 5), lambda i: (i, 0)),
            pl.BlockSpec((5, D), lambda i: (0, 0)),
            pl.BlockSpec((1, D), lambda i: (0, 0)),
            pl.BlockSpec((D, D), lambda i: (0, 0)),
        ],
        out_specs=pl.BlockSpec((BN, D), lambda i: (i, 0)),
        out_shape=jax.ShapeDtypeStruct((N, D), jnp.float32),
    )(x, wl_pad, bl_pad, wc_pad)


def _normalize(h2, deg_parts):
    def body(h2_ref, dp_ref, g_ref, dis_ref):
        deg = dp_ref[0, :] + dp_ref[1, :] + 1.0
        dis = lax.rsqrt(deg)
        g_ref[...] = h2_ref[...] * dis[:, None]
        dis_ref[...] = dis[:, None]
    return pl.pallas_call(
        body,
        grid=(_GRID,),
        in_specs=[
            pl.BlockSpec((BN, D), lambda i: (i, 0)),
            pl.BlockSpec((2, BN), lambda i: (0, i)),
        ],
        out_specs=[
            pl.BlockSpec((BN, D), lambda i: (i, 0)),
            pl.BlockSpec((BN, 1), lambda i: (i, 0)),
        ],
        out_shape=[
            jax.ShapeDtypeStruct((N, D), jnp.float32),
            jax.ShapeDtypeStruct((N, 1), jnp.float32),
        ],
    )(h2, deg_parts)


def _head(acc_parts, g, dis, wo_pad, bc_pad, bo):
    def body(a_ref, g_ref, dis_ref, wo_ref, bc_ref, bo_ref, y_ref):
        a = a_ref[0] + a_ref[1] + g_ref[...]
        pre = a * dis_ref[...] + bc_ref[...]
        y_ref[...] = jnp.dot(jnp.tanh(pre), wo_ref[...],
                             preferred_element_type=jnp.float32) + bo_ref[...]
    return pl.pallas_call(
        body,
        grid=(_GRID,),
        in_specs=[
            pl.BlockSpec((2, BN, D), lambda i: (0, i, 0)),
            pl.BlockSpec((BN, D), lambda i: (i, 0)),
            pl.BlockSpec((BN, 1), lambda i: (i, 0)),
            pl.BlockSpec((D, 1), lambda i: (0, 0)),
            pl.BlockSpec((1, D), lambda i: (0, 0)),
            pl.BlockSpec((1, 1), lambda i: (0, 0)),
        ],
        out_specs=pl.BlockSpec((BN, 1), lambda i: (i, 0)),
        out_shape=jax.ShapeDtypeStruct((N, 1), jnp.float32),
    )(acc_parts, g, dis, wo_pad, bc_pad, bo)


def kernel(x, edge_index, W_lift, b_lift, W_conv, b_conv, W_out, b_out):
    src2d = edge_index[0].reshape(NR, R)
    dst2d = edge_index[1].reshape(NR, R)

    wl_pad = jnp.zeros((5, D), jnp.float32).at[:, :10].set(W_lift.T)
    bl_pad = jnp.zeros((1, D), jnp.float32).at[0, :10].set(b_lift)
    wc_pad = jnp.zeros((D, D), jnp.float32).at[:10, :10].set(W_conv.T)
    bc_pad = jnp.zeros((1, D), jnp.float32).at[0, :10].set(b_conv)
    wo_pad = jnp.zeros((D, 1), jnp.float32).at[:10, 0].set(W_out[0])
    bo = b_out.reshape(1, 1)

    h2 = _lift(x, wl_pad, bl_pad, wc_pad)
    deg_parts = _deg_kernel(dst2d)
    g, dis = _normalize(h2, deg_parts)
    acc_parts = _msg_kernel(src2d, dst2d, g)
    return _head(acc_parts, g, dis, wo_pad, bc_pad, bo)


# speedup vs baseline: 138.3773x; 1.6337x over previous
import functools

import jax
import jax.numpy as jnp
from jax import lax
from jax.experimental import pallas as pl
from jax.experimental.pallas import tpu as pltpu
from jax.experimental.pallas import tpu_sc as plsc

N = 100000
E = 6400000
R = 80
NC, NS = 2, 16
NW = NC * NS
EPW = E // NW
K = 20
C = K * R
T = EPW // C
D = 16
NPAD = 100352
SL = NPAD // NS
FCH = SL // 4

_mesh = plsc.VectorSubcoreMesh(
    core_axis_name="c", subcore_axis_name="s", num_cores=NC, num_subcores=NS)
_sc_params = pltpu.CompilerParams(use_tc_tiling_on_sc=False)


@functools.partial(
    pl.kernel,
    out_type=jax.ShapeDtypeStruct((NC, NPAD), jnp.float32),
    mesh=_mesh,
    compiler_params=_sc_params,
    scratch_types=[
        pltpu.VMEM((C,), jnp.int32),
        pltpu.VMEM((R,), jnp.float32),
        pltpu.VMEM((SL,), jnp.float32),
        pltpu.VMEM_SHARED((NPAD,), jnp.float32),
        pltpu.SemaphoreType.DMA,
    ],
)
def _deg_kernel(dst_hbm, deg_out, ibuf, ones, tbuf, deg_sp, sem):
    c = lax.axis_index("c")
    s = lax.axis_index("s")
    w = s * NC + c
    z = jnp.zeros((16,), jnp.float32)
    o = jnp.ones((16,), jnp.float32)
    for i in range(R // 16):
        ones[pl.ds(i * 16, 16)] = o

    def zb(i, _):
        tbuf[pl.ds(i * 16, 16)] = z
        return _
    lax.fori_loop(0, SL // 16, zb, 0)
    pltpu.sync_copy(tbuf, deg_sp.at[pl.ds(s * SL, SL)])
    plsc.subcore_barrier()

    def step(t, _):
        pltpu.sync_copy(dst_hbm.at[pl.ds(w * EPW + t * C, C)], ibuf)

        def fire(j, _):
            pltpu.async_copy(ones, deg_sp.at[ibuf.at[pl.ds(j * R, R)]],
                             sem, add=True)
            return _
        lax.fori_loop(0, K, fire, 0)

        def drain(j, _):
            pltpu.make_async_copy(ones, deg_sp.at[ibuf.at[pl.ds(j * R, R)]],
                                  sem).wait()
            return _
        lax.fori_loop(0, K, drain, 0)
        return _
    lax.fori_loop(0, T, step, 0)

    plsc.subcore_barrier()
    pltpu.sync_copy(deg_sp.at[pl.ds(s * SL, SL)], tbuf)
    pltpu.sync_copy(tbuf, deg_out.at[c, pl.ds(s * SL, SL)])


@functools.partial(
    pl.kernel,
    out_type=jax.ShapeDtypeStruct((NC, NPAD, D), jnp.float32),
    mesh=_mesh,
    compiler_params=_sc_params,
    scratch_types=[
        pltpu.VMEM((C,), jnp.int32),
        pltpu.VMEM((C,), jnp.int32),
        pltpu.VMEM((C, D), jnp.float32),
        pltpu.VMEM_SHARED((NPAD, D), jnp.float32),
        pltpu.SemaphoreType.DMA,
        pltpu.SemaphoreType.DMA,
    ],
)
def _msg_kernel(src_hbm, dst_hbm, g_hbm, acc_out,
                isrc, idst, rows, acc_sp, sem_g, sem_s):
    c = lax.axis_index("c")
    s = lax.axis_index("s")
    w = s * NC + c
    z = jnp.zeros((16,), jnp.float32)

    def zb(i, _):
        rows[i, :] = z
        return _
    lax.fori_loop(0, FCH, zb, 0)
    for q in range(4):
        pltpu.sync_copy(rows.at[pl.ds(0, FCH)],
                        acc_sp.at[pl.ds(s * SL + q * FCH, FCH)])
    plsc.subcore_barrier()

    def step(t, _):
        base = w * EPW + t * C
        pltpu.sync_copy(src_hbm.at[pl.ds(base, C)], isrc)
        pltpu.sync_copy(dst_hbm.at[pl.ds(base, C)], idst)

        def fire_g(j, _):
            pltpu.async_copy(g_hbm.at[isrc.at[pl.ds(j * R, R)]],
                             rows.at[pl.ds(j * R, R)], sem_g)
            return _
        lax.fori_loop(0, K, fire_g, 0)

        def fire_s(j, _):
            pltpu.make_async_copy(g_hbm.at[isrc.at[pl.ds(j * R, R)]],
                                  rows.at[pl.ds(j * R, R)], sem_g).wait()
            pltpu.async_copy(rows.at[pl.ds(j * R, R)],
                             acc_sp.at[idst.at[pl.ds(j * R, R)]],
                             sem_s, add=True)
            return _
        lax.fori_loop(0, K, fire_s, 0)

        def drain_s(j, _):
            pltpu.make_async_copy(rows.at[pl.ds(j * R, R)],
                                  acc_sp.at[idst.at[pl.ds(j * R, R)]],
                                  sem_s).wait()
            return _
        lax.fori_loop(0, K, drain_s, 0)
        return _
    lax.fori_loop(0, T, step, 0)

    plsc.subcore_barrier()
    for q in range(4):
        pltpu.sync_copy(acc_sp.at[pl.ds(s * SL + q * FCH, FCH)],
                        rows.at[pl.ds(0, FCH)])
        pltpu.sync_copy(rows.at[pl.ds(0, FCH)],
                        acc_out.at[c, pl.ds(s * SL + q * FCH, FCH)])


BN = 8192
_GRID = (N + BN - 1) // BN


def _lift(x, wl_pad, bl_pad, wc_pad):
    def body(x_ref, wl_ref, bl_ref, wc_ref, h2_ref):
        h = jnp.tanh(jnp.dot(x_ref[...], wl_ref[...],
                             preferred_element_type=jnp.float32) + bl_ref[...])
        h2_ref[...] = jnp.dot(h, wc_ref[...],
                              preferred_element_type=jnp.float32)
    return pl.pallas_call(
        body,
        grid=(_GRID,),
        in_specs=[
            pl.BlockSpec((BN, 5), lambda i: (i, 0)),
            pl.BlockSpec((5, D), lambda i: (0, 0)),
            pl.BlockSpec((1, D), lambda i: (0, 0)),
            pl.BlockSpec((D, D), lambda i: (0, 0)),
        ],
        out_specs=pl.BlockSpec((BN, D), lambda i: (i, 0)),
        out_shape=jax.ShapeDtypeStruct((N, D), jnp.float32),
    )(x, wl_pad, bl_pad, wc_pad)


def _normalize(h2, deg_parts):
    def body(h2_ref, dp_ref, g_ref, dis_ref):
        deg = dp_ref[0, :] + dp_ref[1, :] + 1.0
        dis = lax.rsqrt(deg)
        g_ref[...] = h2_ref[...] * dis[:, None]
        dis_ref[...] = dis[:, None]
    return pl.pallas_call(
        body,
        grid=(_GRID,),
        in_specs=[
            pl.BlockSpec((BN, D), lambda i: (i, 0)),
            pl.BlockSpec((2, BN), lambda i: (0, i)),
        ],
        out_specs=[
            pl.BlockSpec((BN, D), lambda i: (i, 0)),
            pl.BlockSpec((BN, 1), lambda i: (i, 0)),
        ],
        out_shape=[
            jax.ShapeDtypeStruct((N, D), jnp.float32),
            jax.ShapeDtypeStruct((N, 1), jnp.float32),
        ],
    )(h2, deg_parts)


def _head(acc_parts, g, dis, wo_pad, bc_pad, bo):
    def body(a_ref, g_ref, dis_ref, wo_ref, bc_ref, bo_ref, y_ref):
        a = a_ref[0] + a_ref[1] + g_ref[...]
        pre = a * dis_ref[...] + bc_ref[...]
        y_ref[...] = jnp.dot(jnp.tanh(pre), wo_ref[...],
                             preferred_element_type=jnp.float32) + bo_ref[...]
    return pl.pallas_call(
        body,
        grid=(_GRID,),
        in_specs=[
            pl.BlockSpec((2, BN, D), lambda i: (0, i, 0)),
            pl.BlockSpec((BN, D), lambda i: (i, 0)),
            pl.BlockSpec((BN, 1), lambda i: (i, 0)),
            pl.BlockSpec((D, 1), lambda i: (0, 0)),
            pl.BlockSpec((1, D), lambda i: (0, 0)),
            pl.BlockSpec((1, 1), lambda i: (0, 0)),
        ],
        out_specs=pl.BlockSpec((BN, 1), lambda i: (i, 0)),
        out_shape=jax.ShapeDtypeStruct((N, 1), jnp.float32),
    )(acc_parts, g, dis, wo_pad, bc_pad, bo)


def kernel(x, edge_index, W_lift, b_lift, W_conv, b_conv, W_out, b_out):
    src1d = edge_index[0]
    dst1d = edge_index[1]

    wl_pad = jnp.zeros((5, D), jnp.float32).at[:, :10].set(W_lift.T)
    bl_pad = jnp.zeros((1, D), jnp.float32).at[0, :10].set(b_lift)
    wc_pad = jnp.zeros((D, D), jnp.float32).at[:10, :10].set(W_conv.T)
    bc_pad = jnp.zeros((1, D), jnp.float32).at[0, :10].set(b_conv)
    wo_pad = jnp.zeros((D, 1), jnp.float32).at[:10, 0].set(W_out[0])
    bo = b_out.reshape(1, 1)

    h2 = _lift(x, wl_pad, bl_pad, wc_pad)
    deg_parts = _deg_kernel(dst1d)
    g, dis = _normalize(h2, deg_parts)
    acc_parts = _msg_kernel(src1d, dst1d, g)
    return _head(acc_parts, g, dis, wo_pad, bc_pad, bo)


# speedup vs baseline: 212.5769x; 1.5362x over previous
import functools

import jax
import jax.numpy as jnp
from jax import lax
from jax.experimental import pallas as pl
from jax.experimental.pallas import tpu as pltpu
from jax.experimental.pallas import tpu_sc as plsc

N = 100000
E = 6400000
R = 80
NC, NS = 2, 16
NW = NC * NS
EPW = E // NW
KD = 20
CD = KD * R
TD = EPW // CD
K = 10
C = K * R
T = EPW // C
D = 16
NPAD = 100352
SL = NPAD // NS
FCH = SL // 4

_mesh = plsc.VectorSubcoreMesh(
    core_axis_name="c", subcore_axis_name="s", num_cores=NC, num_subcores=NS)
_sc_params = pltpu.CompilerParams(use_tc_tiling_on_sc=False)


@functools.partial(
    pl.kernel,
    out_type=jax.ShapeDtypeStruct((NC, NPAD), jnp.float32),
    mesh=_mesh,
    compiler_params=_sc_params,
    scratch_types=[
        pltpu.VMEM((2, CD), jnp.int32),
        pltpu.VMEM((R,), jnp.float32),
        pltpu.VMEM((SL,), jnp.float32),
        pltpu.VMEM_SHARED((NPAD,), jnp.float32),
        pltpu.SemaphoreType.DMA,
        pltpu.SemaphoreType.DMA,
    ],
)
def _deg_kernel(dst_hbm, deg_out, ibuf, ones, tbuf, deg_sp, sem_i, sem_s):
    c = lax.axis_index("c")
    s = lax.axis_index("s")
    w = s * NC + c
    z = jnp.zeros((16,), jnp.float32)
    o = jnp.ones((16,), jnp.float32)
    for i in range(R // 16):
        ones[pl.ds(i * 16, 16)] = o

    def zb(i, _):
        tbuf[pl.ds(i * 16, 16)] = z
        return _
    lax.fori_loop(0, SL // 16, zb, 0)
    pltpu.sync_copy(tbuf, deg_sp.at[pl.ds(s * SL, SL)])
    plsc.subcore_barrier()

    pltpu.sync_copy(dst_hbm.at[pl.ds(w * EPW, CD)], ibuf.at[0])

    def step(t, _):
        p = lax.rem(t, 2)

        @pl.when(t + 1 < TD)
        def _prefetch():
            pltpu.async_copy(dst_hbm.at[pl.ds(w * EPW + (t + 1) * CD, CD)],
                             ibuf.at[1 - p], sem_i)

        def fire(j, _):
            pltpu.async_copy(ones, deg_sp.at[ibuf.at[p, pl.ds(j * R, R)]],
                             sem_s, add=True)
            return _
        lax.fori_loop(0, KD, fire, 0)

        def drain(j, _):
            pltpu.make_async_copy(ones,
                                  deg_sp.at[ibuf.at[p, pl.ds(j * R, R)]],
                                  sem_s).wait()
            return _
        lax.fori_loop(0, KD, drain, 0)

        @pl.when(t + 1 < TD)
        def _wait_prefetch():
            pltpu.make_async_copy(dst_hbm.at[pl.ds(0, CD)], ibuf.at[1 - p],
                                  sem_i).wait()
        return _
    lax.fori_loop(0, TD, step, 0)

    plsc.subcore_barrier()
    pltpu.sync_copy(deg_sp.at[pl.ds(s * SL, SL)], tbuf)
    pltpu.sync_copy(tbuf, deg_out.at[c, pl.ds(s * SL, SL)])


@functools.partial(
    pl.kernel,
    out_type=[jax.ShapeDtypeStruct((NPAD, D), jnp.float32),
              jax.ShapeDtypeStruct((NPAD, D), jnp.float32)],
    mesh=_mesh,
    compiler_params=_sc_params,
    scratch_types=[
        pltpu.VMEM((2, C), jnp.int32),
        pltpu.VMEM((2, C), jnp.int32),
        pltpu.VMEM((2 * C, D), jnp.float32),
        pltpu.VMEM_SHARED((NPAD, D), jnp.float32),
        pltpu.SemaphoreType.DMA,
        pltpu.SemaphoreType.DMA,
        pltpu.SemaphoreType.DMA,
    ],
)
def _msg_kernel(src_hbm, dst_hbm, g_hbm, acc0_out, acc1_out,
                isrc, idst, rows, acc_sp, sem_i, sem_g, sem_s):
    c = lax.axis_index("c")
    s = lax.axis_index("s")
    w = s * NC + c
    z = jnp.zeros((16,), jnp.float32)

    def zb(i, _):
        rows[i, :] = z
        return _
    lax.fori_loop(0, FCH, zb, 0)
    for q in range(4):
        pltpu.sync_copy(rows.at[pl.ds(0, FCH)],
                        acc_sp.at[pl.ds(s * SL + q * FCH, FCH)])
    plsc.subcore_barrier()

    def fire_gathers(p, t):
        def fire_g(j, _):
            pltpu.async_copy(g_hbm.at[isrc.at[p, pl.ds(j * R, R)]],
                             rows.at[pl.ds(p * C + j * R, R)], sem_g)
            return _
        lax.fori_loop(0, K, fire_g, 0)

    pltpu.sync_copy(src_hbm.at[pl.ds(w * EPW, C)], isrc.at[0])
    pltpu.sync_copy(dst_hbm.at[pl.ds(w * EPW, C)], idst.at[0])
    fire_gathers(0, 0)

    def step(t, _):
        p = lax.rem(t, 2)

        @pl.when(t + 1 < T)
        def _prefetch_idx():
            base = w * EPW + (t + 1) * C
            pltpu.async_copy(src_hbm.at[pl.ds(base, C)], isrc.at[1 - p],
                             sem_i)
            pltpu.async_copy(dst_hbm.at[pl.ds(base, C)], idst.at[1 - p],
                             sem_i)

        def fire_s(j, _):
            pltpu.make_async_copy(g_hbm.at[isrc.at[p, pl.ds(j * R, R)]],
                                  rows.at[pl.ds(p * C + j * R, R)],
                                  sem_g).wait()
            pltpu.async_copy(rows.at[pl.ds(p * C + j * R, R)],
                             acc_sp.at[idst.at[p, pl.ds(j * R, R)]],
                             sem_s, add=True)
            return _
        lax.fori_loop(0, K, fire_s, 0)

        @pl.when(t + 1 < T)
        def _next_gathers():
            pltpu.make_async_copy(src_hbm.at[pl.ds(0, C)], isrc.at[1 - p],
                                  sem_i).wait()
            pltpu.make_async_copy(dst_hbm.at[pl.ds(0, C)], idst.at[1 - p],
                                  sem_i).wait()
            fire_gathers(1 - p, t + 1)

        def drain_s(j, _):
            pltpu.make_async_copy(rows.at[pl.ds(p * C + j * R, R)],
                                  acc_sp.at[idst.at[p, pl.ds(j * R, R)]],
                                  sem_s).wait()
            return _
        lax.fori_loop(0, K, drain_s, 0)
        return _
    lax.fori_loop(0, T, step, 0)

    plsc.subcore_barrier()
    for q in range(4):
        pltpu.sync_copy(acc_sp.at[pl.ds(s * SL + q * FCH, FCH)],
                        rows.at[pl.ds(0, FCH)])

        @pl.when(c == 0)
        def _flush0():
            pltpu.sync_copy(rows.at[pl.ds(0, FCH)],
                            acc0_out.at[pl.ds(s * SL + q * FCH, FCH)])

        @pl.when(c == 1)
        def _flush1():
            pltpu.sync_copy(rows.at[pl.ds(0, FCH)],
                            acc1_out.at[pl.ds(s * SL + q * FCH, FCH)])


BNW = 1024
_GRID = (N // 8 + BNW - 1) // BNW


def _lift(xw, wl8, bl8, wc8):
    def body(x_ref, wl_ref, bl_ref, wc_ref, h2_ref):
        h = jnp.tanh(jnp.dot(x_ref[...], wl_ref[...],
                             preferred_element_type=jnp.float32) + bl_ref[...])
        h2_ref[...] = jnp.dot(h, wc_ref[...],
                              preferred_element_type=jnp.float32)
    return pl.pallas_call(
        body,
        grid=(_GRID,),
        in_specs=[
            pl.BlockSpec((BNW, 40), lambda i: (i, 0)),
            pl.BlockSpec((40, 128), lambda i: (0, 0)),
            pl.BlockSpec((1, 128), lambda i: (0, 0)),
            pl.BlockSpec((128, 128), lambda i: (0, 0)),
        ],
        out_specs=pl.BlockSpec((BNW, 128), lambda i: (i, 0)),
        out_shape=jax.ShapeDtypeStruct((N // 8, 128), jnp.float32),
    )(xw, wl8, bl8, wc8)


def _normalize(h2w, deg8, expand):
    def body(h2_ref, d8_ref, ex_ref, g_ref, dis_ref):
        deg = d8_ref[0] + d8_ref[1] + 1.0
        dis8 = lax.rsqrt(deg)
        dis16 = jnp.dot(dis8, ex_ref[...],
                        preferred_element_type=jnp.float32)
        g_ref[...] = h2_ref[...] * dis16
        dis_ref[...] = dis16
    return pl.pallas_call(
        body,
        grid=(_GRID,),
        in_specs=[
            pl.BlockSpec((BNW, 128), lambda i: (i, 0)),
            pl.BlockSpec((2, BNW, 8), lambda i: (0, i, 0)),
            pl.BlockSpec((8, 128), lambda i: (0, 0)),
        ],
        out_specs=[
            pl.BlockSpec((BNW, 128), lambda i: (i, 0)),
            pl.BlockSpec((BNW, 128), lambda i: (i, 0)),
        ],
        out_shape=[
            jax.ShapeDtypeStruct((N // 8, 128), jnp.float32),
            jax.ShapeDtypeStruct((N // 8, 128), jnp.float32),
        ],
    )(h2w, deg8, expand)


def _head(acc0w, acc1w, gw, dis16w, wo8, bc8, bo8):
    def body(a0_ref, a1_ref, g_ref, dis_ref, wo_ref, bc_ref, bo_ref, y_ref):
        a = a0_ref[...] + a1_ref[...] + g_ref[...]
        pre = a * dis_ref[...] + bc_ref[...]
        y_ref[...] = jnp.dot(jnp.tanh(pre), wo_ref[...],
                             preferred_element_type=jnp.float32) + bo_ref[...]
    return pl.pallas_call(
        body,
        grid=(_GRID,),
        in_specs=[
            pl.BlockSpec((BNW, 128), lambda i: (i, 0)),
            pl.BlockSpec((BNW, 128), lambda i: (i, 0)),
            pl.BlockSpec((BNW, 128), lambda i: (i, 0)),
            pl.BlockSpec((BNW, 128), lambda i: (i, 0)),
            pl.BlockSpec((128, 8), lambda i: (0, 0)),
            pl.BlockSpec((1, 128), lambda i: (0, 0)),
            pl.BlockSpec((1, 8), lambda i: (0, 0)),
        ],
        out_specs=pl.BlockSpec((BNW, 8), lambda i: (i, 0)),
        out_shape=jax.ShapeDtypeStruct((N // 8, 8), jnp.float32),
    )(acc0w, acc1w, gw, dis16w, wo8, bc8, bo8)


def kernel(x, edge_index, W_lift, b_lift, W_conv, b_conv, W_out, b_out):
    src1d = edge_index[0]
    dst1d = edge_index[1]

    z = jnp.zeros
    wl8 = z((40, 128), jnp.float32)
    bl8 = z((1, 128), jnp.float32)
    wc8 = z((128, 128), jnp.float32)
    bc8 = z((1, 128), jnp.float32)
    wo8 = z((128, 8), jnp.float32)
    expand = z((8, 128), jnp.float32)
    for b in range(8):
        wl8 = wl8.at[5 * b:5 * b + 5, 16 * b:16 * b + 10].set(W_lift.T)
        bl8 = bl8.at[0, 16 * b:16 * b + 10].set(b_lift)
        wc8 = wc8.at[16 * b:16 * b + 10, 16 * b:16 * b + 10].set(W_conv.T)
        bc8 = bc8.at[0, 16 * b:16 * b + 10].set(b_conv)
        wo8 = wo8.at[16 * b:16 * b + 10, b].set(W_out[0])
        expand = expand.at[b, 16 * b:16 * b + 16].set(1.0)
    bo8 = jnp.broadcast_to(b_out.reshape(1, 1), (1, 8))

    xw = x.reshape(N // 8, 40)
    h2w = _lift(xw, wl8, bl8, wc8)
    deg_parts = _deg_kernel(dst1d)
    gw, dis16w = _normalize(h2w, deg_parts.reshape(2, NPAD // 8, 8), expand)
    acc0, acc1 = _msg_kernel(src1d, dst1d, gw.reshape(N, D))
    y8 = _head(acc0.reshape(NPAD // 8, 128), acc1.reshape(NPAD // 8, 128),
               gw, dis16w, wo8, bc8, bo8)
    return y8.reshape(N, 1)


# speedup vs baseline: 218.2256x; 1.0266x over previous
import functools

import jax
import jax.numpy as jnp
from jax import lax
from jax.experimental import pallas as pl
from jax.experimental.pallas import tpu as pltpu
from jax.experimental.pallas import tpu_sc as plsc

N = 100000
E = 6400000
R = 80
NC, NS = 2, 16
NW = NC * NS
EPW = E // NW
KD = 20
CD = KD * R
TD = EPW // CD
K = 10
C = K * R
T = EPW // C
D = 16
NPAD = 100352
SL = NPAD // NS
FCH = SL // 4

_mesh = plsc.VectorSubcoreMesh(
    core_axis_name="c", subcore_axis_name="s", num_cores=NC, num_subcores=NS)
_sc_params = pltpu.CompilerParams(use_tc_tiling_on_sc=False)


@functools.partial(
    pl.kernel,
    out_type=jax.ShapeDtypeStruct((NC, NPAD), jnp.float32),
    mesh=_mesh,
    compiler_params=_sc_params,
    scratch_types=[
        pltpu.VMEM((2, CD), jnp.int32),
        pltpu.VMEM((R,), jnp.float32),
        pltpu.VMEM((SL,), jnp.float32),
        pltpu.VMEM_SHARED((NPAD,), jnp.float32),
        pltpu.SemaphoreType.DMA,
        pltpu.SemaphoreType.DMA,
    ],
)
def _deg_kernel(edges_hbm, deg_out, ibuf, ones, tbuf, deg_sp, sem_i, sem_s):
    c = lax.axis_index("c")
    s = lax.axis_index("s")
    w = s * NC + c
    z = jnp.zeros((16,), jnp.float32)
    o = jnp.ones((16,), jnp.float32)
    for i in range(R // 16):
        ones[pl.ds(i * 16, 16)] = o

    def zb(i, _):
        tbuf[pl.ds(i * 16, 16)] = z
        return _
    lax.fori_loop(0, SL // 16, zb, 0)
    pltpu.sync_copy(tbuf, deg_sp.at[pl.ds(s * SL, SL)])
    plsc.subcore_barrier()

    pltpu.sync_copy(edges_hbm.at[pl.ds(E + w * EPW, CD)], ibuf.at[0])

    def step(t, _):
        p = lax.rem(t, 2)

        @pl.when(t + 1 < TD)
        def _prefetch():
            pltpu.async_copy(
                edges_hbm.at[pl.ds(E + w * EPW + (t + 1) * CD, CD)],
                ibuf.at[1 - p], sem_i)

        def fire(j, _):
            pltpu.async_copy(ones, deg_sp.at[ibuf.at[p, pl.ds(j * R, R)]],
                             sem_s, add=True)
            return _
        lax.fori_loop(0, KD, fire, 0)

        def drain(j, _):
            pltpu.make_async_copy(ones,
                                  deg_sp.at[ibuf.at[p, pl.ds(j * R, R)]],
                                  sem_s).wait()
            return _
        lax.fori_loop(0, KD, drain, 0)

        @pl.when(t + 1 < TD)
        def _wait_prefetch():
            pltpu.make_async_copy(edges_hbm.at[pl.ds(0, CD)],
                                  ibuf.at[1 - p], sem_i).wait()
        return _
    lax.fori_loop(0, TD, step, 0)

    plsc.subcore_barrier()
    pltpu.sync_copy(deg_sp.at[pl.ds(s * SL, SL)], tbuf)
    pltpu.sync_copy(tbuf, deg_out.at[c, pl.ds(s * SL, SL)])


@functools.partial(
    pl.kernel,
    out_type=[jax.ShapeDtypeStruct((NPAD, D), jnp.float32),
              jax.ShapeDtypeStruct((NPAD, D), jnp.float32)],
    mesh=_mesh,
    compiler_params=_sc_params,
    scratch_types=[
        pltpu.VMEM((2, C), jnp.int32),
        pltpu.VMEM((2, C), jnp.int32),
        pltpu.VMEM((2 * C, D), jnp.float32),
        pltpu.VMEM_SHARED((NPAD, D), jnp.float32),
        pltpu.SemaphoreType.DMA,
        pltpu.SemaphoreType.DMA,
        pltpu.SemaphoreType.DMA,
    ],
)
def _msg_kernel(edges_hbm, g_hbm, acc0_out, acc1_out,
                isrc, idst, rows, acc_sp, sem_i, sem_g, sem_s):
    c = lax.axis_index("c")
    s = lax.axis_index("s")
    w = s * NC + c
    z = jnp.zeros((16,), jnp.float32)

    def zb(i, _):
        rows[i, :] = z
        return _
    lax.fori_loop(0, FCH, zb, 0)
    for q in range(4):
        pltpu.sync_copy(rows.at[pl.ds(0, FCH)],
                        acc_sp.at[pl.ds(s * SL + q * FCH, FCH)])
    plsc.subcore_barrier()

    def fire_gathers(p, t):
        def fire_g(j, _):
            pltpu.async_copy(g_hbm.at[isrc.at[p, pl.ds(j * R, R)]],
                             rows.at[pl.ds(p * C + j * R, R)], sem_g)
            return _
        lax.fori_loop(0, K, fire_g, 0)

    pltpu.sync_copy(edges_hbm.at[pl.ds(w * EPW, C)], isrc.at[0])
    pltpu.sync_copy(edges_hbm.at[pl.ds(E + w * EPW, C)], idst.at[0])
    fire_gathers(0, 0)

    def step(t, _):
        p = lax.rem(t, 2)

        @pl.when(t + 1 < T)
        def _prefetch_idx():
            base = w * EPW + (t + 1) * C
            pltpu.async_copy(edges_hbm.at[pl.ds(base, C)], isrc.at[1 - p],
                             sem_i)
            pltpu.async_copy(edges_hbm.at[pl.ds(E + base, C)],
                             idst.at[1 - p], sem_i)

        def fire_s(j, _):
            pltpu.make_async_copy(g_hbm.at[isrc.at[p, pl.ds(j * R, R)]],
                                  rows.at[pl.ds(p * C + j * R, R)],
                                  sem_g).wait()
            pltpu.async_copy(rows.at[pl.ds(p * C + j * R, R)],
                             acc_sp.at[idst.at[p, pl.ds(j * R, R)]],
                             sem_s, add=True)
            return _
        lax.fori_loop(0, K, fire_s, 0)

        @pl.when(t + 1 < T)
        def _next_gathers():
            pltpu.make_async_copy(edges_hbm.at[pl.ds(0, C)], isrc.at[1 - p],
                                  sem_i).wait()
            pltpu.make_async_copy(edges_hbm.at[pl.ds(0, C)], idst.at[1 - p],
                                  sem_i).wait()
            fire_gathers(1 - p, t + 1)

        def drain_s(j, _):
            pltpu.make_async_copy(rows.at[pl.ds(p * C + j * R, R)],
                                  acc_sp.at[idst.at[p, pl.ds(j * R, R)]],
                                  sem_s).wait()
            return _
        lax.fori_loop(0, K, drain_s, 0)
        return _
    lax.fori_loop(0, T, step, 0)

    plsc.subcore_barrier()
    for q in range(4):
        pltpu.sync_copy(acc_sp.at[pl.ds(s * SL + q * FCH, FCH)],
                        rows.at[pl.ds(0, FCH)])

        @pl.when(c == 0)
        def _flush0():
            pltpu.sync_copy(rows.at[pl.ds(0, FCH)],
                            acc0_out.at[pl.ds(s * SL + q * FCH, FCH)])

        @pl.when(c == 1)
        def _flush1():
            pltpu.sync_copy(rows.at[pl.ds(0, FCH)],
                            acc1_out.at[pl.ds(s * SL + q * FCH, FCH)])


BNW = 1024
_GRID = (N // 8 + BNW - 1) // BNW


def _lift(xw, wl8, bl8, wc8):
    def body(x_ref, wl_ref, bl_ref, wc_ref, h2_ref):
        h = jnp.tanh(jnp.dot(x_ref[...], wl_ref[...],
                             preferred_element_type=jnp.float32) + bl_ref[...])
        h2_ref[...] = jnp.dot(h, wc_ref[...],
                              preferred_element_type=jnp.float32)
    return pl.pallas_call(
        body,
        grid=(_GRID,),
        in_specs=[
            pl.BlockSpec((BNW, 40), lambda i: (i, 0)),
            pl.BlockSpec((40, 128), lambda i: (0, 0)),
            pl.BlockSpec((1, 128), lambda i: (0, 0)),
            pl.BlockSpec((128, 128), lambda i: (0, 0)),
        ],
        out_specs=pl.BlockSpec((BNW, 128), lambda i: (i, 0)),
        out_shape=jax.ShapeDtypeStruct((N // 8, 128), jnp.float32),
    )(xw, wl8, bl8, wc8)


def _normalize(h2w, deg8, expand):
    def body(h2_ref, d8_ref, ex_ref, g_ref, dis_ref):
        deg = d8_ref[0] + d8_ref[1] + 1.0
        dis8 = lax.rsqrt(deg)
        dis16 = jnp.dot(dis8, ex_ref[...],
                        preferred_element_type=jnp.float32)
        g_ref[...] = h2_ref[...] * dis16
        dis_ref[...] = dis16
    return pl.pallas_call(
        body,
        grid=(_GRID,),
        in_specs=[
            pl.BlockSpec((BNW, 128), lambda i: (i, 0)),
            pl.BlockSpec((2, BNW, 8), lambda i: (0, i, 0)),
            pl.BlockSpec((8, 128), lambda i: (0, 0)),
        ],
        out_specs=[
            pl.BlockSpec((BNW, 128), lambda i: (i, 0)),
            pl.BlockSpec((BNW, 128), lambda i: (i, 0)),
        ],
        out_shape=[
            jax.ShapeDtypeStruct((N // 8, 128), jnp.float32),
            jax.ShapeDtypeStruct((N // 8, 128), jnp.float32),
        ],
    )(h2w, deg8, expand)


def _head(acc0w, acc1w, gw, dis16w, wo8, bc8, bo8):
    def body(a0_ref, a1_ref, g_ref, dis_ref, wo_ref, bc_ref, bo_ref, y_ref):
        a = a0_ref[...] + a1_ref[...] + g_ref[...]
        pre = a * dis_ref[...] + bc_ref[...]
        y_ref[...] = jnp.dot(jnp.tanh(pre), wo_ref[...],
                             preferred_element_type=jnp.float32) + bo_ref[...]
    return pl.pallas_call(
        body,
        grid=(_GRID,),
        in_specs=[
            pl.BlockSpec((BNW, 128), lambda i: (i, 0)),
            pl.BlockSpec((BNW, 128), lambda i: (i, 0)),
            pl.BlockSpec((BNW, 128), lambda i: (i, 0)),
            pl.BlockSpec((BNW, 128), lambda i: (i, 0)),
            pl.BlockSpec((128, 8), lambda i: (0, 0)),
            pl.BlockSpec((1, 128), lambda i: (0, 0)),
            pl.BlockSpec((1, 8), lambda i: (0, 0)),
        ],
        out_specs=pl.BlockSpec((BNW, 8), lambda i: (i, 0)),
        out_shape=jax.ShapeDtypeStruct((N // 8, 8), jnp.float32),
    )(acc0w, acc1w, gw, dis16w, wo8, bc8, bo8)


def kernel(x, edge_index, W_lift, b_lift, W_conv, b_conv, W_out, b_out):
    eflat = edge_index.reshape(2 * E)

    z = jnp.zeros
    wl8 = z((40, 128), jnp.float32)
    bl8 = z((1, 128), jnp.float32)
    wc8 = z((128, 128), jnp.float32)
    bc8 = z((1, 128), jnp.float32)
    wo8 = z((128, 8), jnp.float32)
    expand = z((8, 128), jnp.float32)
    for b in range(8):
        wl8 = wl8.at[5 * b:5 * b + 5, 16 * b:16 * b + 10].set(W_lift.T)
        bl8 = bl8.at[0, 16 * b:16 * b + 10].set(b_lift)
        wc8 = wc8.at[16 * b:16 * b + 10, 16 * b:16 * b + 10].set(W_conv.T)
        bc8 = bc8.at[0, 16 * b:16 * b + 10].set(b_conv)
        wo8 = wo8.at[16 * b:16 * b + 10, b].set(W_out[0])
        expand = expand.at[b, 16 * b:16 * b + 16].set(1.0)
    bo8 = jnp.broadcast_to(b_out.reshape(1, 1), (1, 8))

    xw = x.reshape(N // 8, 40)
    h2w = _lift(xw, wl8, bl8, wc8)
    deg_parts = _deg_kernel(eflat)
    gw, dis16w = _normalize(h2w, deg_parts.reshape(2, NPAD // 8, 8), expand)
    acc0, acc1 = _msg_kernel(eflat, gw.reshape(N, D))
    y8 = _head(acc0.reshape(NPAD // 8, 128), acc1.reshape(NPAD // 8, 128),
               gw, dis16w, wo8, bc8, bo8)
    return y8.reshape(N, 1)
